# baseline (speedup 1.0000x reference)
import functools

import jax
import jax.numpy as jnp
from jax import lax
from jax.experimental import pallas as pl
from jax.experimental.pallas import tpu as pltpu
from jax.experimental.pallas import tpu_sc as plsc

PI = 3.141592653589793
DIM = 128
DIM_R = DIM // 2
LANES = 16
NC, NS = 2, 16
NW = NC * NS
CHUNK = 80
UNROLL = 4


def _rotate_body(x_ref, rel_ref, rot_ref):
    x = x_ref[...]
    re = x[:, :DIM_R]
    im = x[:, DIM_R:]
    r = rel_ref[0, :] / PI
    c = jnp.cos(r)
    s = jnp.sin(r)
    rot_ref[:, :DIM_R] = re * c - im * s
    rot_ref[:, DIM_R:] = im * c + re * s


def _rotate_table(x, rel):
    return pl.pallas_call(
        _rotate_body,
        out_shape=jax.ShapeDtypeStruct(x.shape, jnp.float32),
    )(x, rel)


def _soft_sqrt(a):
    nha = a * (-0.5)
    i = plsc.bitcast(a, jnp.int32)
    i = jnp.int32(0x5F3759DF) - lax.shift_right_logical(i, 1)
    y = plsc.bitcast(i, jnp.float32)
    y = y * (1.5 + nha * y * y)
    return a * y


def _sc_body(rot_hbm, x_hbm, u_hbm, v_hbm, out_hbm,
             idxu, idxv, out_all, ru0, rv0, ru1, rv1, ru2, rv2,
             su0, sv0, su1, sv1, su2, sv2):
    wid = lax.axis_index("s") * NC + lax.axis_index("c")
    n_per_w = out_hbm.shape[0] // NW
    n_chunks = n_per_w // CHUNK
    base_w = wid * n_per_w
    lane = lax.iota(jnp.int32, LANES)

    pltpu.sync_copy(u_hbm.at[pl.ds(base_w, n_per_w)], idxu)
    pltpu.sync_copy(v_hbm.at[pl.ds(base_w, n_per_w)], idxv)

    def start(ci, ru, rv, su, sv):
        iu = idxu.at[pl.ds(ci * CHUNK, CHUNK)]
        iv = idxv.at[pl.ds(ci * CHUNK, CHUNK)]
        pltpu.async_copy(rot_hbm.at[iu], ru, su)
        pltpu.async_copy(x_hbm.at[iv], rv, sv)

    def wait(ru, rv, su, sv):
        iu = idxu.at[pl.ds(0, CHUNK)]
        iv = idxv.at[pl.ds(0, CHUNK)]
        pltpu.make_async_copy(rot_hbm.at[iu], ru, su).wait()
        pltpu.make_async_copy(x_hbm.at[iv], rv, sv).wait()

    def compute(ci, ru, rv):
        base = ci * CHUNK

        @plsc.parallel_loop(0, CHUNK // LANES)
        def _(g):
            scores = jnp.zeros((LANES,), jnp.float32)
            for e_loc in range(LANES):
                e = g * LANES + e_loc
                acc = jnp.zeros((LANES,), jnp.float32)
                for k in range(DIM_R // LANES):
                    dr = (ru[e, pl.ds(k * LANES, LANES)]
                          - rv[e, pl.ds(k * LANES, LANES)])
                    di = (ru[e, pl.ds(DIM_R + k * LANES, LANES)]
                          - rv[e, pl.ds(DIM_R + k * LANES, LANES)])
                    acc = acc + _soft_sqrt(dr * dr + di * di)
                scores = jnp.where(lane == e_loc, jnp.sum(acc), scores)
            out_all[pl.ds(base + g * LANES, LANES)] = scores

    bufs = ((ru0, rv0, su0, sv0), (ru1, rv1, su1, sv1), (ru2, rv2, su2, sv2))
    for b in range(3):
        start(b, *bufs[b])

    def tri_body(i, _):
        c0 = 3 * i
        for b in range(3):
            wait(*bufs[b])
            start(c0 + b + 3, *bufs[b])
            compute(c0 + b, bufs[b][0], bufs[b][1])
        return ()

    t = n_chunks - 5
    lax.fori_loop(0, t // 3, tri_body, ())
    wait(*bufs[0])
    compute(t, bufs[0][0], bufs[0][1])
    start(t + 3, *bufs[0])
    wait(*bufs[1])
    compute(t + 1, bufs[1][0], bufs[1][1])
    start(t + 4, *bufs[1])
    wait(*bufs[2])
    compute(t + 2, bufs[2][0], bufs[2][1])
    wait(*bufs[0])
    compute(t + 3, bufs[0][0], bufs[0][1])
    wait(*bufs[1])
    compute(t + 4, bufs[1][0], bufs[1][1])

    pltpu.sync_copy(out_all, out_hbm.at[pl.ds(base_w, n_per_w)])


def _edge_scores(rot, x, u_idx, v_idx, n_edges):
    n_per_w = n_edges // NW
    assert n_edges % NW == 0 and n_per_w % CHUNK == 0
    assert (n_per_w // CHUNK) % 3 == 2 and n_per_w // CHUNK >= 5
    mesh = plsc.VectorSubcoreMesh(core_axis_name="c", subcore_axis_name="s")
    f = functools.partial(
        pl.kernel,
        out_type=jax.ShapeDtypeStruct((n_edges,), jnp.float32),
        mesh=mesh,
        scratch_types=[
            pltpu.VMEM((n_per_w,), jnp.int32),
            pltpu.VMEM((n_per_w,), jnp.int32),
            pltpu.VMEM((n_per_w,), jnp.float32),
            pltpu.VMEM((CHUNK, DIM), jnp.float32),
            pltpu.VMEM((CHUNK, DIM), jnp.float32),
            pltpu.VMEM((CHUNK, DIM), jnp.float32),
            pltpu.VMEM((CHUNK, DIM), jnp.float32),
            pltpu.VMEM((CHUNK, DIM), jnp.float32),
            pltpu.VMEM((CHUNK, DIM), jnp.float32),
            pltpu.SemaphoreType.DMA,
            pltpu.SemaphoreType.DMA,
            pltpu.SemaphoreType.DMA,
            pltpu.SemaphoreType.DMA,
            pltpu.SemaphoreType.DMA,
            pltpu.SemaphoreType.DMA,
        ],
        compiler_params=pltpu.CompilerParams(needs_layout_passes=False),
    )(_sc_body)
    return f(rot, x, u_idx, v_idx)


def kernel(x, edge_index, rel):
    n_edges = edge_index.shape[1]
    u_idx = edge_index[0].astype(jnp.int32)
    v_idx = edge_index[1].astype(jnp.int32)
    rot = _rotate_table(x, rel)
    return _edge_scores(rot, x, u_idx, v_idx, n_edges)
